# baseline (speedup 1.0000x reference)
import jax
import jax.numpy as jnp
from jax import lax
from jax.experimental import pallas as pl
from jax.experimental.pallas import tpu as pltpu
from jax.experimental.pallas import tpu_sc as plsc

BATCH = 16384
SEQ = 200
D = 64
V = 50
NC, NS = 2, 16
NW = NC * NS
K = BATCH // NW
L = 16


def _lookup_body(idxt_hbm, ttab_hbm, out_hbm,
                 ttab_v, idx0, idx1, stage0, stage1,
                 sem_t, sem_i0, sem_i1, sem_o0, sem_o1):
    wid = lax.axis_index("s") * NC + lax.axis_index("c")
    b0 = wid * K
    lanes = lax.iota(jnp.int32, L)
    idx_v = (idx0, idx1)
    stage_v = (stage0, stage1)
    sem_i = (sem_i0, sem_i1)
    sem_o = (sem_o0, sem_o1)

    pltpu.async_copy(ttab_hbm, ttab_v, sem_t).wait()

    def start_idx(s, b):
        pltpu.async_copy(idxt_hbm.at[s, pl.ds(b0, K)], idx_v[b], sem_i[b])

    start_idx(0, 0)
    start_idx(1, 1)

    def outer(s2, carry):
        for b in range(2):
            s = 2 * s2 + b
            pltpu.make_async_copy(idxt_hbm.at[0, pl.ds(0, K)],
                                  idx_v[b], sem_i[b]).wait()

            @pl.when(s2 > 0)
            def _():
                pltpu.make_async_copy(
                    stage_v[b], out_hbm.at[0, :, pl.ds(0, K)], sem_o[b]).wait()

            @plsc.parallel_loop(0, K // L, unroll=6)
            def lane_group(t):
                v0 = plsc.load_gather(idx_v[b], [lanes + L * t])
                for d in range(D):
                    stage_v[b][d, pl.ds(L * t, L)] = plsc.load_gather(
                        ttab_v, [v0 + jnp.int32(d * D)])

            @pl.when(s2 < SEQ // 2 - 1)
            def _():
                start_idx(s + 2, b)

            pltpu.async_copy(stage_v[b], out_hbm.at[s, :, pl.ds(b0, K)],
                             sem_o[b])
        return carry

    lax.fori_loop(0, SEQ // 2, outer, 0)
    for b in range(2):
        pltpu.make_async_copy(stage_v[b],
                              out_hbm.at[0, :, pl.ds(0, K)], sem_o[b]).wait()


@jax.jit
def _lookup(pos_ids, table):
    idxt = pos_ids.T
    ttab_flat = jnp.pad(table, ((0, D - V), (0, 0))).T.reshape(D * D)

    mesh = plsc.VectorSubcoreMesh(
        core_axis_name="c", subcore_axis_name="s", num_cores=NC, num_subcores=NS
    )
    out_t = pl.kernel(
        _lookup_body,
        out_type=jax.ShapeDtypeStruct((SEQ, D, BATCH), jnp.float32),
        mesh=mesh,
        compiler_params=pltpu.CompilerParams(needs_layout_passes=False),
        scratch_types=[
            pltpu.VMEM((D * D,), jnp.float32),
            pltpu.VMEM((K,), jnp.int32),
            pltpu.VMEM((K,), jnp.int32),
            pltpu.VMEM((D, K), jnp.float32),
            pltpu.VMEM((D, K), jnp.float32),
        ] + [pltpu.SemaphoreType.DMA] * 5,
    )(idxt, ttab_flat)
    return out_t.transpose(2, 0, 1)


def kernel(pos_ids, table):
    return _lookup(pos_ids, table)


# speedup vs baseline: 1.9580x; 1.9580x over previous
import jax
import jax.numpy as jnp
from jax import lax
from jax.experimental import pallas as pl
from jax.experimental.pallas import tpu as pltpu
from jax.experimental.pallas import tpu_sc as plsc

BATCH = 16384
SEQ = 200
D = 64
V = 50
NC, NS = 2, 16
NW = NC * NS
NQ, NO = 4, 8
KQ = BATCH // NQ
DO = D // NO
L = 16


def _lookup_body(idxt_hbm, ttab_hbm, out_hbm,
                 ttab_v, idx0, idx1, stage0, stage1,
                 sem_t, sem_i0, sem_i1, sem_o0, sem_o1):
    wid = lax.axis_index("s") * NC + lax.axis_index("c")
    q = wid // NO
    o = wid % NO
    b0 = q * KQ
    d0 = o * DO
    lanes = lax.iota(jnp.int32, L)
    idx_v = (idx0, idx1)
    stage_v = (stage0, stage1)
    sem_i = (sem_i0, sem_i1)
    sem_o = (sem_o0, sem_o1)

    pltpu.async_copy(ttab_hbm, ttab_v, sem_t).wait()

    def start_idx(s, b):
        pltpu.async_copy(idxt_hbm.at[s, pl.ds(b0, KQ)], idx_v[b], sem_i[b])

    start_idx(0, 0)
    start_idx(1, 1)

    def outer(s2, carry):
        for b in range(2):
            s = 2 * s2 + b
            pltpu.make_async_copy(idxt_hbm.at[0, pl.ds(0, KQ)],
                                  idx_v[b], sem_i[b]).wait()

            @pl.when(s2 > 0)
            def _():
                pltpu.make_async_copy(
                    stage_v[b], out_hbm.at[0, pl.ds(0, DO), pl.ds(0, KQ)],
                    sem_o[b]).wait()

            @plsc.parallel_loop(0, KQ // L, unroll=4)
            def lane_group(t):
                v0 = plsc.load_gather(idx_v[b], [lanes + L * t]) + d0 * D
                for dd in range(DO):
                    stage_v[b][dd, pl.ds(L * t, L)] = plsc.load_gather(
                        ttab_v, [v0 + jnp.int32(dd * D)])

            @pl.when(s2 < SEQ // 2 - 1)
            def _():
                start_idx(s + 2, b)

            pltpu.async_copy(
                stage_v[b], out_hbm.at[s, pl.ds(d0, DO), pl.ds(b0, KQ)],
                sem_o[b])
        return carry

    lax.fori_loop(0, SEQ // 2, outer, 0)
    for b in range(2):
        pltpu.make_async_copy(stage_v[b],
                              out_hbm.at[0, pl.ds(0, DO), pl.ds(0, KQ)],
                              sem_o[b]).wait()


@jax.jit
def _lookup(pos_ids, table):
    idxt = pos_ids.T
    ttab_flat = jnp.pad(table, ((0, D - V), (0, 0))).T.reshape(D * D)

    mesh = plsc.VectorSubcoreMesh(
        core_axis_name="c", subcore_axis_name="s", num_cores=NC, num_subcores=NS
    )
    out_t = pl.kernel(
        _lookup_body,
        out_type=jax.ShapeDtypeStruct((SEQ, D, BATCH), jnp.float32),
        mesh=mesh,
        compiler_params=pltpu.CompilerParams(needs_layout_passes=False),
        scratch_types=[
            pltpu.VMEM((D * D,), jnp.float32),
            pltpu.VMEM((KQ,), jnp.int32),
            pltpu.VMEM((KQ,), jnp.int32),
            pltpu.VMEM((DO, KQ), jnp.float32),
            pltpu.VMEM((DO, KQ), jnp.float32),
        ] + [pltpu.SemaphoreType.DMA] * 5,
    )(idxt, ttab_flat)
    return out_t.transpose(2, 0, 1)


def kernel(pos_ids, table):
    return _lookup(pos_ids, table)
